# baseline (speedup 1.0000x reference)
import functools

import jax
import jax.numpy as jnp
from jax import lax
from jax.experimental import pallas as pl
from jax.experimental.pallas import tpu as pltpu
from jax.experimental.pallas import tpu_sc as plsc

BATCH = 16384
NS = 5
NFACT = 4
NRULES = NS ** NFACT

_info = plsc.get_sparse_core_info()
_NC, _NSUB, _L = _info.num_cores, _info.num_subcores, _info.num_lanes
NW = _NC * _NSUB
ROWS_PER_W = BATCH // NW
RB = 16
NBLK = ROWS_PER_W // RB
MT_W = NFACT * NS * ROWS_PER_W
NG_C = 39


def _sc_call(mt):
    mesh = plsc.VectorSubcoreMesh(core_axis_name="c", subcore_axis_name="s")

    @functools.partial(
        pl.kernel,
        mesh=mesh,
        out_type=jax.ShapeDtypeStruct((BATCH, NRULES), jnp.float32),
        compiler_params=pltpu.CompilerParams(needs_layout_passes=False),
        scratch_types=[
            pltpu.VMEM((MT_W,), jnp.float32),
            pltpu.VMEM((32,), jnp.float32),
            pltpu.VMEM((128,), jnp.float32),
            pltpu.VMEM((2 * RB, NRULES), jnp.float32),
            pltpu.SemaphoreType.DMA,
            pltpu.SemaphoreType.DMA,
        ],
    )
    def k(mt_hbm, out_hbm, mt_v, w01_v, w012_v, buf_v, sem0, sem1):
        wid = lax.axis_index("s") * _NC + lax.axis_index("c")
        pltpu.sync_copy(mt_hbm.at[pl.ds(wid * MT_W, MT_W)], mt_v)
        lane = lax.iota(jnp.int32, _L)

        pa = [jnp.minimum(lane + 16 * g, 24) for g in range(2)]
        a0 = [(p // 5) * ROWS_PER_W for p in pa]
        a1 = [(NS + p % 5) * ROWS_PER_W for p in pa]
        pb = [jnp.minimum(lane + 16 * g, 124) for g in range(8)]
        bia = [p // 5 for p in pb]
        bib = [(2 * NS + p % 5) * ROWS_PER_W for p in pb]
        cdiv = [(lane + m) // 5 for m in range(5)]
        csmod = [((lane + m) % 5) * ROWS_PER_W for m in range(5)]
        m3_base = 3 * NS * ROWS_PER_W

        def _drain(sem):
            for _ in range(2):
                pltpu.make_async_copy(buf_v.at[pl.ds(0, 8)],
                                      out_hbm.at[pl.ds(0, 8)], sem).wait()

        def block(t, carry):
            par = jnp.bitwise_and(t, 1)
            bro = par * RB

            @pl.when(t >= 2)
            def _():
                @pl.when(par == 0)
                def _():
                    _drain(sem0)
                @pl.when(par == 1)
                def _():
                    _drain(sem1)

            row_t = t * RB
            for l in range(RB):
                row = row_t + l
                bigrow = bro + l
                for g in range(2):
                    a = plsc.load_gather(mt_v, [a0[g] + row])
                    b = plsc.load_gather(mt_v, [a1[g] + row])
                    w01_v[pl.ds(16 * g, 16)] = a * b
                for g in range(8):
                    a = plsc.load_gather(w01_v, [bia[g]])
                    b = plsc.load_gather(mt_v, [bib[g] + row])
                    w012_v[pl.ds(16 * g, 16)] = a * b
                ib = [plsc.load_gather(mt_v, [csmod[m] + (m3_base + row)])
                      for m in range(5)]
                for g in range(NG_C):
                    m = g % 5
                    a = plsc.load_gather(w012_v, [cdiv[m] + (3 * g + g // 5)])
                    buf_v[bigrow, pl.ds(16 * g, 16)] = a * ib[m]
            c4 = [mt_v[pl.ds((j * NS + 4) * ROWS_PER_W + row_t, RB)]
                  for j in range(NFACT)]
            v624 = (c4[0] * c4[1]) * (c4[2] * c4[3])
            plsc.store_scatter(
                buf_v, [lane + bro, jnp.full((_L,), NRULES - 1, jnp.int32)],
                v624)
            row0 = wid * ROWS_PER_W + row_t

            @pl.when(par == 0)
            def _():
                pltpu.async_copy(buf_v.at[pl.ds(0, 8)],
                                 out_hbm.at[pl.ds(row0, 8)], sem0)
                pltpu.async_copy(buf_v.at[pl.ds(8, 8)],
                                 out_hbm.at[pl.ds(row0 + 8, 8)], sem0)

            @pl.when(par == 1)
            def _():
                pltpu.async_copy(buf_v.at[pl.ds(RB, 8)],
                                 out_hbm.at[pl.ds(row0, 8)], sem1)
                pltpu.async_copy(buf_v.at[pl.ds(RB + 8, 8)],
                                 out_hbm.at[pl.ds(row0 + 8, 8)], sem1)
            return carry

        lax.fori_loop(0, NBLK, block, 0)
        _drain(sem0)
        _drain(sem1)

    return k(mt)


def kernel(m0, m1, m2, m3):
    mt = jnp.concatenate([m0.T, m1.T, m2.T, m3.T], axis=0)
    mt = mt.reshape(NFACT * NS, NW, ROWS_PER_W).transpose(1, 0, 2)
    return _sc_call(mt.reshape(-1))


# speedup vs baseline: 2.3562x; 2.3562x over previous
import functools

import jax
import jax.numpy as jnp
from jax import lax
from jax.experimental import pallas as pl
from jax.experimental.pallas import tpu as pltpu
from jax.experimental.pallas import tpu_sc as plsc

BATCH = 16384
NS = 5
NFACT = 4
NRULES = NS ** NFACT

_info = plsc.get_sparse_core_info()
_NC, _NSUB, _L = _info.num_cores, _info.num_subcores, _info.num_lanes
NW = _NC * _NSUB
ROWS_PER_W = BATCH // NW
RB = 16
NBLK = ROWS_PER_W // RB
MT_W = NFACT * NS * ROWS_PER_W
BUF_W = RB * NRULES


def _sc_call(mt):
    mesh = plsc.VectorSubcoreMesh(core_axis_name="c", subcore_axis_name="s")

    @functools.partial(
        pl.kernel,
        mesh=mesh,
        out_type=jax.ShapeDtypeStruct((BATCH * NRULES,), jnp.float32),
        compiler_params=pltpu.CompilerParams(needs_layout_passes=False),
        scratch_types=[
            pltpu.VMEM((MT_W,), jnp.float32),
            pltpu.VMEM((2 * BUF_W,), jnp.float32),
            pltpu.SemaphoreType.DMA,
            pltpu.SemaphoreType.DMA,
        ],
    )
    def k(mt_hbm, out_hbm, mt_v, buf_v, sem0, sem1):
        wid = lax.axis_index("s") * _NC + lax.axis_index("c")
        pltpu.sync_copy(mt_hbm.at[pl.ds(wid * MT_W, MT_W)], mt_v)
        lane_off = lax.iota(jnp.int32, _L) * NRULES

        def _drain(sem):
            pltpu.make_async_copy(buf_v.at[pl.ds(0, BUF_W)],
                                  out_hbm.at[pl.ds(0, BUF_W)], sem).wait()

        def block(t, carry):
            par = jnp.bitwise_and(t, 1)
            base_idx = lane_off + par * BUF_W

            @pl.when(t >= 2)
            def _():
                @pl.when(par == 0)
                def _():
                    _drain(sem0)
                @pl.when(par == 1)
                def _():
                    _drain(sem1)

            vs = [[mt_v[pl.ds((j * NS + i) * ROWS_PER_W + t * RB, RB)]
                   for i in range(NS)] for j in range(NFACT)]
            for i0 in range(NS):
                v0 = vs[0][i0]
                for i1 in range(NS):
                    v01 = v0 * vs[1][i1]
                    for i2 in range(NS):
                        v012 = v01 * vs[2][i2]
                        for i3 in range(NS):
                            r = ((i0 * NS + i1) * NS + i2) * NS + i3
                            val = v012 * vs[3][i3]
                            plsc.store_scatter(buf_v, [base_idx + r], val)
            out_off = (wid * ROWS_PER_W + t * RB) * NRULES

            @pl.when(par == 0)
            def _():
                pltpu.async_copy(buf_v.at[pl.ds(0, BUF_W)],
                                 out_hbm.at[pl.ds(out_off, BUF_W)], sem0)

            @pl.when(par == 1)
            def _():
                pltpu.async_copy(buf_v.at[pl.ds(BUF_W, BUF_W)],
                                 out_hbm.at[pl.ds(out_off, BUF_W)], sem1)
            return carry

        lax.fori_loop(0, NBLK, block, 0)
        _drain(sem0)
        _drain(sem1)

    return k(mt)


def kernel(m0, m1, m2, m3):
    mt = jnp.concatenate([m0.T, m1.T, m2.T, m3.T], axis=0)
    mt = mt.reshape(NFACT * NS, NW, ROWS_PER_W).transpose(1, 0, 2)
    return _sc_call(mt.reshape(-1)).reshape(BATCH, NRULES)
